# baseline (speedup 1.0000x reference)
import functools

import jax
import jax.numpy as jnp
from jax import lax
from jax.experimental import pallas as pl
from jax.experimental.pallas import tpu as pltpu
from jax.experimental.pallas import tpu_sc as plsc

_NC = 2
_NS = 16
_NW = _NC * _NS

_NBUF = 2
_GB = 128


def _body(xt_hbm, table_hbm, out_hbm, idx_v, rows_v, outt_v, *sems):
  n = xt_hbm.shape[0]
  d = table_hbm.shape[1]
  ngrp = n // _GB // _NW
  nlap = ngrp // _NBUF
  gsems = sems[:_NBUF]
  ssems = sems[_NBUF:]

  wid = lax.axis_index("s") * _NC + lax.axis_index("c")
  base_g = wid * ngrp

  pltpu.sync_copy(xt_hbm.at[pl.ds(base_g * _GB, ngrp * _GB)], idx_v)

  iota16 = lax.iota(jnp.int32, 16)

  def gather(g, b):
    return pltpu.make_async_copy(
        table_hbm.at[idx_v.at[pl.ds(g * _GB, _GB)]], rows_v.at[b], gsems[b])

  def store(g, b):
    gw = base_g + g
    h = gw // 128
    bt = gw % 128
    return pltpu.make_async_copy(outt_v.at[b], out_hbm.at[h, :, bt],
                                 ssems[b])

  def transpose(b):
    for dt in range(d // 8):
      for di in range(8):
        col = jnp.full((16,), 8 * dt + di, jnp.int32)
        for k in range(_GB // 16):
          vals = plsc.load_gather(rows_v.at[b], [iota16 + 16 * k, col])
          outt_v[b, dt, di, pl.ds(16 * k, 16)] = vals

  for b in range(_NBUF):
    gather(b, b).start()
  for b in range(_NBUF):
    gather(b, b).wait()
    transpose(b)
    store(b, b).start()
    gather(b + _NBUF, b).start()

  @pl.loop(1, nlap - 1)
  def _(lap):
    g0 = lap * _NBUF
    for b in range(_NBUF):
      g = g0 + b
      gather(g, b).wait()
      store(g - _NBUF, b).wait()
      transpose(b)
      store(g, b).start()
      gather(g + _NBUF, b).start()

  g0 = ngrp - _NBUF
  for b in range(_NBUF):
    g = g0 + b
    gather(g, b).wait()
    store(g - _NBUF, b).wait()
    transpose(b)
    store(g, b).start()
  for b in range(_NBUF):
    store(g0 + b, b).wait()


def _run(xt_flat, table):
  n = xt_flat.shape[0]
  d = table.shape[1]
  nh = n // 16384
  per_w = n // _NW
  mesh = plsc.VectorSubcoreMesh(core_axis_name="c", subcore_axis_name="s")
  sems = [pltpu.SemaphoreType.DMA] * (2 * _NBUF)
  return pl.kernel(
      _body,
      out_type=jax.ShapeDtypeStruct((nh, d // 8, 128, 8, 128), table.dtype),
      mesh=mesh,
      compiler_params=pltpu.CompilerParams(use_tc_tiling_on_sc=False,
                                           needs_layout_passes=False),
      scratch_types=[
          pltpu.VMEM((per_w,), jnp.int32),
          pltpu.VMEM((_NBUF, _GB, d), table.dtype),
          pltpu.VMEM((_NBUF, d // 8, 8, 128), table.dtype),
      ] + sems,
  )(xt_flat, table)


@jax.jit
def kernel(x, table):
  b, h = x.shape
  d = table.shape[1]
  xt = jnp.transpose(x).reshape(b * h).astype(jnp.int32)
  out5 = _run(xt, table)
  return jnp.transpose(out5, (2, 4, 0, 1, 3)).reshape(b, h, d)


# speedup vs baseline: 1.0644x; 1.0644x over previous
import functools

import jax
import jax.numpy as jnp
from jax import lax
from jax.experimental import pallas as pl
from jax.experimental.pallas import tpu as pltpu
from jax.experimental.pallas import tpu_sc as plsc

_NC = 2
_NS = 16
_NW = _NC * _NS

_NBUF = 4
_GB = 128


def _body(xt_hbm, table_hbm, out_hbm, idx_v, rows_v, outt_v, *sems):
  n = xt_hbm.shape[0]
  d = table_hbm.shape[1]
  ngrp = n // _GB // _NW
  nlap = ngrp // _NBUF
  gsems = sems[:_NBUF]
  ssems = sems[_NBUF:]

  wid = lax.axis_index("s") * _NC + lax.axis_index("c")
  base_g = wid * ngrp

  pltpu.sync_copy(xt_hbm.at[pl.ds(base_g * _GB, ngrp * _GB)], idx_v)

  iota16 = lax.iota(jnp.int32, 16)

  def gather(g, b):
    return pltpu.make_async_copy(
        table_hbm.at[idx_v.at[pl.ds(g * _GB, _GB)]], rows_v.at[b], gsems[b])

  def store(g, b):
    gw = base_g + g
    h = gw // 128
    bt = gw % 128
    return pltpu.make_async_copy(outt_v.at[b], out_hbm.at[h, :, bt],
                                 ssems[b])

  def transpose(b):
    @pl.loop(0, d // 8)
    def _(dt):
      for di in range(8):
        col = jnp.full((16,), 8 * dt + di, jnp.int32)
        for k in range(_GB // 16):
          vals = plsc.load_gather(rows_v.at[b], [iota16 + 16 * k, col])
          outt_v[b, dt, di, pl.ds(16 * k, 16)] = vals

  for b in range(_NBUF):
    gather(b, b).start()

  @pl.loop(0, nlap)
  def _(lap):
    g0 = lap * _NBUF
    for b in range(_NBUF):
      g = g0 + b
      gather(g, b).wait()

      @pl.when(lap > 0)
      def _():
        store(g - _NBUF, b).wait()

      transpose(b)
      store(g, b).start()

      @pl.when(lap < nlap - 1)
      def _():
        gather(g + _NBUF, b).start()

  for b in range(_NBUF):
    store(ngrp - _NBUF + b, b).wait()


def _run(xt_flat, table):
  n = xt_flat.shape[0]
  d = table.shape[1]
  nh = n // 16384
  per_w = n // _NW
  mesh = plsc.VectorSubcoreMesh(core_axis_name="c", subcore_axis_name="s")
  sems = [pltpu.SemaphoreType.DMA] * (2 * _NBUF)
  return pl.kernel(
      _body,
      out_type=jax.ShapeDtypeStruct((nh, d // 8, 128, 8, 128), table.dtype),
      mesh=mesh,
      compiler_params=pltpu.CompilerParams(use_tc_tiling_on_sc=False,
                                           needs_layout_passes=False),
      scratch_types=[
          pltpu.VMEM((per_w,), jnp.int32),
          pltpu.VMEM((_NBUF, _GB, d), table.dtype),
          pltpu.VMEM((_NBUF, d // 8, 8, 128), table.dtype),
      ] + sems,
  )(xt_flat, table)


@jax.jit
def kernel(x, table):
  b, h = x.shape
  d = table.shape[1]
  xt = jnp.transpose(x).reshape(b * h).astype(jnp.int32)
  out5 = _run(xt, table)
  return jnp.transpose(out5, (2, 4, 0, 1, 3)).reshape(b, h, d)


# speedup vs baseline: 2.0474x; 1.9235x over previous
import functools

import jax
import jax.numpy as jnp
from jax import lax
from jax.experimental import pallas as pl
from jax.experimental.pallas import tpu as pltpu
from jax.experimental.pallas import tpu_sc as plsc

_NC = 2
_NS = 16
_NW = _NC * _NS

_NBUF = 4
_GB = 128


def _body(xt_hbm, table_hbm, out_hbm, idx_v, rows_v, outt_v, *sems):
  n = xt_hbm.shape[0]
  d = table_hbm.shape[1]
  ngrp = n // _GB // _NW
  nlap = ngrp // _NBUF
  gsems = sems[:_NBUF]
  ssems = sems[_NBUF:]

  wid = lax.axis_index("s") * _NC + lax.axis_index("c")
  base_g = wid * ngrp

  pltpu.sync_copy(xt_hbm.at[pl.ds(base_g * _GB, ngrp * _GB)], idx_v)

  iota16 = lax.iota(jnp.int32, 16)

  def gather(g, b):
    return pltpu.make_async_copy(
        table_hbm.at[idx_v.at[pl.ds(g * _GB, _GB)]], rows_v.at[b], gsems[b])

  def store(g, b):
    gw = base_g + g
    h = gw // 128
    bt = gw % 128
    return pltpu.make_async_copy(outt_v.at[b, :, :, pl.ds(0, _GB)],
                                 out_hbm.at[h, :, bt], ssems[b])

  dtv = [(iota16 >> 3) + 2 * k for k in range(d // 16)]
  div = iota16 & 7

  def transpose(b):
    @pl.loop(0, _GB, unroll=8)
    def _(row):
      biv = jnp.full((16,), row, jnp.int32)
      for k in range(d // 16):
        vals = rows_v[b, row, pl.ds(16 * k, 16)]
        plsc.store_scatter(outt_v.at[b], [dtv[k], div, biv], vals)

  for b in range(_NBUF):
    gather(b, b).start()

  @pl.loop(0, nlap)
  def _(lap):
    g0 = lap * _NBUF
    for b in range(_NBUF):
      g = g0 + b
      gather(g, b).wait()

      @pl.when(lap > 0)
      def _():
        store(g - _NBUF, b).wait()

      transpose(b)
      store(g, b).start()

      @pl.when(lap < nlap - 1)
      def _():
        gather(g + _NBUF, b).start()

  for b in range(_NBUF):
    store(ngrp - _NBUF + b, b).wait()


def _run(xt_flat, table):
  n = xt_flat.shape[0]
  d = table.shape[1]
  nh = n // 16384
  per_w = n // _NW
  mesh = plsc.VectorSubcoreMesh(core_axis_name="c", subcore_axis_name="s")
  sems = [pltpu.SemaphoreType.DMA] * (2 * _NBUF)
  return pl.kernel(
      _body,
      out_type=jax.ShapeDtypeStruct((nh, d // 8, 128, 8, 128), table.dtype),
      mesh=mesh,
      compiler_params=pltpu.CompilerParams(use_tc_tiling_on_sc=False,
                                           needs_layout_passes=False),
      scratch_types=[
          pltpu.VMEM((per_w,), jnp.int32),
          pltpu.VMEM((_NBUF, _GB, d), table.dtype),
          pltpu.VMEM((_NBUF, d // 8, 8, 129), table.dtype),
      ] + sems,
  )(xt_flat, table)


@jax.jit
def kernel(x, table):
  b, h = x.shape
  d = table.shape[1]
  xt = jnp.transpose(x).reshape(b * h).astype(jnp.int32)
  out5 = _run(xt, table)
  return jnp.transpose(out5, (2, 4, 0, 1, 3)).reshape(b, h, d)


# speedup vs baseline: 2.0505x; 1.0015x over previous
import functools

import jax
import jax.numpy as jnp
from jax import lax
from jax.experimental import pallas as pl
from jax.experimental.pallas import tpu as pltpu
from jax.experimental.pallas import tpu_sc as plsc

_NC = 2
_NS = 16
_NW = _NC * _NS

_NBUF = 4
_GB = 128


def _body(xt_hbm, table_hbm, out_hbm, idx_v, rows_v, outt_v, *sems):
  n = xt_hbm.shape[0]
  d = out_hbm.shape[1] * 8
  ngrp = n // _GB // _NW
  nlap = ngrp // _NBUF
  gsems = sems[:_NBUF]
  ssems = sems[_NBUF:]

  wid = lax.axis_index("s") * _NC + lax.axis_index("c")
  base_g = wid * ngrp

  pltpu.sync_copy(xt_hbm.at[pl.ds(base_g * _GB, ngrp * _GB)], idx_v)

  iota16 = lax.iota(jnp.int32, 16)

  def gather(g, b):
    return pltpu.make_async_copy(
        table_hbm.at[idx_v.at[pl.ds(g * _GB, _GB)]], rows_v.at[b], gsems[b])

  def store(g, b):
    gw = base_g + g
    h = gw // 128
    bt = gw % 128
    return pltpu.make_async_copy(outt_v.at[b, :, :, pl.ds(0, _GB)],
                                 out_hbm.at[h, :, bt], ssems[b])

  dtv = [(iota16 >> 3) + 2 * k for k in range(d // 16)]
  div = iota16 & 7

  def transpose(b):
    @pl.loop(0, _GB, unroll=8)
    def _(row):
      biv = jnp.full((16,), row, jnp.int32)
      for k in range(d // 16):
        vals = rows_v[b, row, pl.ds(16 * k, 16)]
        plsc.store_scatter(outt_v.at[b], [dtv[k], div, biv], vals)

  for b in range(_NBUF):
    gather(b, b).start()

  @pl.loop(0, nlap)
  def _(lap):
    g0 = lap * _NBUF
    for b in range(_NBUF):
      g = g0 + b
      gather(g, b).wait()

      @pl.when(lap > 0)
      def _():
        store(g - _NBUF, b).wait()

      transpose(b)
      store(g, b).start()

      @pl.when(lap < nlap - 1)
      def _():
        gather(g + _NBUF, b).start()

  for b in range(_NBUF):
    store(ngrp - _NBUF + b, b).wait()


def _run(xt_flat, tpad, d):
  n = xt_flat.shape[0]
  nh = n // 16384
  per_w = n // _NW
  mesh = plsc.VectorSubcoreMesh(core_axis_name="c", subcore_axis_name="s")
  sems = [pltpu.SemaphoreType.DMA] * (2 * _NBUF)
  return pl.kernel(
      _body,
      out_type=jax.ShapeDtypeStruct((nh, d // 8, 128, 8, 128), tpad.dtype),
      mesh=mesh,
      compiler_params=pltpu.CompilerParams(use_tc_tiling_on_sc=False,
                                           needs_layout_passes=False),
      scratch_types=[
          pltpu.VMEM((per_w,), jnp.int32),
          pltpu.VMEM((_NBUF, _GB, tpad.shape[1]), tpad.dtype),
          pltpu.VMEM((_NBUF, d // 8, 8, 129), tpad.dtype),
      ] + sems,
  )(xt_flat, tpad)


@jax.jit
def kernel(x, table):
  b, h = x.shape
  d = table.shape[1]
  xt = jnp.transpose(x).reshape(b * h).astype(jnp.int32)
  out5 = _run(xt, table, d)
  return jnp.transpose(out5, (2, 4, 0, 1, 3)).reshape(b, h, d)
